# baseline (speedup 1.0000x reference)
import functools
import math

import jax
import jax.numpy as jnp
from jax import lax
from jax.experimental import pallas as pl
from jax.experimental.pallas import tpu as pltpu
from jax.experimental.pallas import tpu_sc as plsc

D_MODEL = 64
SCALE = math.sqrt(D_MODEL)
CHUNK = 128


@functools.lru_cache(maxsize=None)
def _make_sc_lookup(V: int, B: int, L: int, D: int):
    N = B * L
    info = plsc.get_sparse_core_info()
    NC, NS, LN = info.num_cores, info.num_subcores, info.num_lanes
    NW = NC * NS
    assert N % (NW * CHUNK) == 0 and D % LN == 0 and B % CHUNK == 0
    per_w = N // (NW * CHUNK)
    n_btiles = B // CHUNK

    mesh = plsc.VectorSubcoreMesh(core_axis_name="c", subcore_axis_name="s")

    @functools.partial(
        pl.kernel,
        mesh=mesh,
        out_type=jax.ShapeDtypeStruct((L, D, B), jnp.float32),
        scratch_types=[
            pltpu.VMEM((per_w, CHUNK), jnp.int32),
            pltpu.VMEM((CHUNK,), jnp.int32),
            pltpu.VMEM((CHUNK, 128), jnp.float32),
            pltpu.VMEM((D, CHUNK), jnp.float32),
            pltpu.SemaphoreType.DMA,
        ],
        compiler_params=pltpu.CompilerParams(
            use_tc_tiling_on_sc=True, needs_layout_passes=False
        ),
    )
    def lookup(xf_hbm, tp_hbm, out_hbm, idx_v, pk_v, gbuf, sbuf, sem):
        wid = lax.axis_index("s") * NC + lax.axis_index("c")
        cid0 = wid * per_w
        pltpu.sync_copy(xf_hbm.at[pl.ds(cid0, per_w)], idx_v)

        def chunk_body(t, carry):
            cid = cid0 + t
            l = cid // n_btiles
            bt = cid % n_btiles

            for g in range(CHUNK // LN):
                sl = pl.ds(g * LN, LN)
                pk_v[sl] = lax.shift_right_logical(idx_v[t, sl], 1)

            pltpu.async_copy(tp_hbm.at[pk_v], gbuf, sem).wait()

            for g in range(CHUNK // LN):
                sl = pl.ds(g * LN, LN)
                cvec = lax.iota(jnp.int32, LN) + g * LN
                hv = lax.shift_left(idx_v[t, sl] & 1, 6)

                def d_body(d, carry2):
                    cvec2, hv2 = carry2
                    col = hv2 + d
                    val = plsc.load_gather(gbuf, [cvec2, col])
                    sbuf[d, sl] = val * SCALE
                    return (cvec2, hv2)

                lax.fori_loop(0, D, d_body, (cvec, hv), unroll=4)

            pltpu.sync_copy(
                sbuf, out_hbm.at[l, :, pl.ds(bt * CHUNK, CHUNK)]
            )
            return carry

        lax.fori_loop(0, per_w, chunk_body, 0)

    return lookup


def kernel(x, table):
    B, L = x.shape
    V, D = table.shape
    N = B * L
    xf = x.T.reshape(N // CHUNK, CHUNK).astype(jnp.int32)
    tp = table.reshape(V // 2, 2 * D)
    o = _make_sc_lookup(V, B, L, D)(xf, tp)
    return o.transpose(2, 0, 1)


# speedup vs baseline: 1.2120x; 1.2120x over previous
import functools
import math

import jax
import jax.numpy as jnp
from jax import lax
from jax.experimental import pallas as pl
from jax.experimental.pallas import tpu as pltpu
from jax.experimental.pallas import tpu_sc as plsc

D_MODEL = 64
SCALE = math.sqrt(D_MODEL)
CHUNK = 128
NBUF = 2


@functools.lru_cache(maxsize=None)
def _make_sc_lookup(V: int, B: int, L: int, D: int):
    N = B * L
    info = plsc.get_sparse_core_info()
    NC, NS, LN = info.num_cores, info.num_subcores, info.num_lanes
    NW = NC * NS
    assert N % (NW * CHUNK) == 0 and D % LN == 0 and B % CHUNK == 0
    per_w = N // (NW * CHUNK)
    assert per_w % NBUF == 0
    n_btiles = B // CHUNK

    mesh = plsc.VectorSubcoreMesh(core_axis_name="c", subcore_axis_name="s")

    @functools.partial(
        pl.kernel,
        mesh=mesh,
        out_type=jax.ShapeDtypeStruct((L, D, B), jnp.float32),
        scratch_types=[
            pltpu.VMEM((per_w, CHUNK), jnp.int32),
            pltpu.VMEM((per_w, CHUNK), jnp.int32),
            pltpu.VMEM((NBUF, CHUNK, 2 * D), jnp.float32),
            pltpu.VMEM((NBUF, D, CHUNK), jnp.float32),
            pltpu.VMEM((LN, LN), jnp.int32),
        ]
        + [pltpu.SemaphoreType.DMA] * (2 * NBUF),
        compiler_params=pltpu.CompilerParams(
            use_tc_tiling_on_sc=True, needs_layout_passes=False
        ),
    )
    def lookup(xf_hbm, tp_hbm, out_hbm, idx_v, pk_v, gbuf, sbuf, pm_v, *sems):
        gsems = sems[:NBUF]
        ssems = sems[NBUF:]
        wid = lax.axis_index("s") * NC + lax.axis_index("c")
        cid0 = wid * per_w
        pltpu.sync_copy(xf_hbm.at[pl.ds(cid0, per_w)], idx_v)

        def pk_row(t, c):
            for g in range(CHUNK // LN):
                sl = pl.ds(g * LN, LN)
                pk_v[t, sl] = lax.shift_right_logical(idx_v[t, sl], 1)
            return c

        lax.fori_loop(0, per_w, pk_row, 0)

        for b in range(NBUF):
            pltpu.async_copy(tp_hbm.at[pk_v.at[b]], gbuf.at[b], gsems[b])

        lane = lax.iota(jnp.int32, LN)
        for k in range(LN):
            pm_v[k] = (lane + k) & (LN - 1)

        def outer(r, carry):
            g0 = r * NBUF
            for b in range(NBUF):
                j = g0 + b
                cid = cid0 + j
                l = cid // n_btiles
                bt = cid % n_btiles

                pltpu.make_async_copy(
                    tp_hbm.at[pk_v.at[j]], gbuf.at[b], gsems[b]
                ).wait()

                @pl.when(r > 0)
                def _drain():
                    pltpu.make_async_copy(
                        sbuf.at[b],
                        out_hbm.at[0, :, pl.ds(0, CHUNK)],
                        ssems[b],
                    ).wait()

                def g_body(g, c1):
                    sl = pl.ds(g * LN, LN)
                    cvec = lane + g * LN
                    hv = lax.shift_left(idx_v[j, sl] & 1, 6)

                    def d_body(dblk, c2):
                        base = dblk * LN
                        for k in range(LN):
                            dv = pm_v[k] + base
                            col = hv + dv
                            val = plsc.load_gather(gbuf.at[b], [cvec, col])
                            plsc.store_scatter(
                                sbuf.at[b], [dv, cvec], val * SCALE
                            )
                        return c2

                    lax.fori_loop(0, D // LN, d_body, 0)
                    return c1

                lax.fori_loop(0, CHUNK // LN, g_body, 0)

                pltpu.async_copy(
                    sbuf.at[b],
                    out_hbm.at[l, :, pl.ds(bt * CHUNK, CHUNK)],
                    ssems[b],
                )

                @pl.when(j + NBUF < per_w)
                def _refill():
                    pltpu.async_copy(
                        tp_hbm.at[pk_v.at[j + NBUF]], gbuf.at[b], gsems[b]
                    )
            return carry

        lax.fori_loop(0, per_w // NBUF, outer, 0)

        for b in range(NBUF):
            pltpu.make_async_copy(
                sbuf.at[b], out_hbm.at[0, :, pl.ds(0, CHUNK)], ssems[b]
            ).wait()

    return lookup


def kernel(x, table):
    B, L = x.shape
    V, D = table.shape
    N = B * L
    xf = x.T.reshape(N // CHUNK, CHUNK).astype(jnp.int32)
    tp = table.reshape(V // 2, 2 * D)
    o = _make_sc_lookup(V, B, L, D)(xf, tp)
    return o.transpose(2, 0, 1)


# speedup vs baseline: 1.7394x; 1.4351x over previous
import functools
import math

import jax
import jax.numpy as jnp
from jax import lax
from jax.experimental import pallas as pl
from jax.experimental.pallas import tpu as pltpu
from jax.experimental.pallas import tpu_sc as plsc

D_MODEL = 64
SCALE = math.sqrt(D_MODEL)
CHUNK = 128
NBUF = 4


@functools.lru_cache(maxsize=None)
def _make_sc_lookup(V: int, N: int, D: int):
    info = plsc.get_sparse_core_info()
    NC, NS, LN = info.num_cores, info.num_subcores, info.num_lanes
    NW = NC * NS
    assert N % (NW * CHUNK) == 0 and D % LN == 0
    n_chunks = N // (NW * CHUNK)
    assert n_chunks % NBUF == 0

    mesh = plsc.VectorSubcoreMesh(core_axis_name="c", subcore_axis_name="s")

    @functools.partial(
        pl.kernel,
        mesh=mesh,
        out_type=jax.ShapeDtypeStruct((N, 2 * D), jnp.float32),
        scratch_types=[
            pltpu.VMEM((n_chunks, CHUNK), jnp.int32),
            pltpu.VMEM((NBUF, CHUNK, 2 * D), jnp.float32),
        ]
        + [pltpu.SemaphoreType.DMA] * (2 * NBUF),
        compiler_params=pltpu.CompilerParams(
            use_tc_tiling_on_sc=False, needs_layout_passes=False
        ),
    )
    def lookup(xf_hbm, tp_hbm, out_hbm, idx_v, gbuf, *sems):
        gsems = sems[:NBUF]
        ssems = sems[NBUF:]
        wid = lax.axis_index("s") * NC + lax.axis_index("c")
        chunk0 = wid * n_chunks
        row0 = chunk0 * CHUNK
        pltpu.sync_copy(xf_hbm.at[pl.ds(chunk0, n_chunks)], idx_v)

        LAG = 2
        for b in range(LAG):
            pltpu.async_copy(tp_hbm.at[idx_v.at[b]], gbuf.at[b], gsems[b])

        def outer(r, carry):
            g0 = r * NBUF
            for b in range(NBUF):
                j = g0 + b
                pltpu.make_async_copy(
                    tp_hbm.at[idx_v.at[j]], gbuf.at[b], gsems[b]
                ).wait()

                pltpu.async_copy(
                    gbuf.at[b],
                    out_hbm.at[pl.ds(row0 + j * CHUNK, CHUNK)],
                    ssems[b],
                )

                bf = (b + LAG) % NBUF

                def _wait_store():
                    pltpu.make_async_copy(
                        gbuf.at[bf],
                        out_hbm.at[pl.ds(row0, CHUNK)],
                        ssems[bf],
                    ).wait()

                if b >= NBUF - LAG:
                    _wait_store()
                else:
                    pl.when(r > 0)(_wait_store)

                @pl.when(j + LAG < n_chunks)
                def _refill():
                    pltpu.async_copy(
                        tp_hbm.at[idx_v.at[j + LAG]], gbuf.at[bf], gsems[bf]
                    )
            return carry

        lax.fori_loop(0, n_chunks // NBUF, outer, 0)

        for j in range(n_chunks - LAG, n_chunks):
            b = j % NBUF
            pltpu.make_async_copy(
                gbuf.at[b], out_hbm.at[pl.ds(row0, CHUNK)], ssems[b]
            ).wait()

    return lookup


def kernel(x, table):
    B, L = x.shape
    V, D = table.shape
    N = B * L
    xf = x.reshape(N // CHUNK, CHUNK).astype(jnp.int32)
    tp = jnp.pad(table * jnp.float32(SCALE), ((0, 0), (0, D)))
    o = _make_sc_lookup(V, N, D)(xf, tp)
    return o[:, :D].reshape(B, L, D)


# speedup vs baseline: 2.4324x; 1.3984x over previous
import functools
import math

import jax
import jax.numpy as jnp
from jax import lax
from jax.experimental import pallas as pl
from jax.experimental.pallas import tpu as pltpu
from jax.experimental.pallas import tpu_sc as plsc

D_MODEL = 64
SCALE = math.sqrt(D_MODEL)
CHUNK = 128
NBUF = 4


@functools.lru_cache(maxsize=None)
def _make_sc_lookup(V: int, N: int, D: int):
    info = plsc.get_sparse_core_info()
    NC, NS, LN = info.num_cores, info.num_subcores, info.num_lanes
    NW = NC * NS
    assert N % (NW * CHUNK) == 0 and D % LN == 0
    n_chunks = N // (NW * CHUNK)
    assert n_chunks % NBUF == 0

    mesh = plsc.VectorSubcoreMesh(core_axis_name="c", subcore_axis_name="s")

    @functools.partial(
        pl.kernel,
        mesh=mesh,
        out_type=jax.ShapeDtypeStruct((N, 2 * D), jnp.float32),
        scratch_types=[
            pltpu.VMEM((n_chunks, CHUNK), jnp.int32),
            pltpu.VMEM((NBUF, CHUNK, D), jnp.float32),
        ]
        + [pltpu.SemaphoreType.DMA] * (2 * NBUF),
        compiler_params=pltpu.CompilerParams(
            use_tc_tiling_on_sc=False, needs_layout_passes=False
        ),
    )
    def lookup(xf_hbm, tp_hbm, out_hbm, idx_v, gbuf, *sems):
        gsems = sems[:NBUF]
        ssems = sems[NBUF:]
        wid = lax.axis_index("s") * NC + lax.axis_index("c")
        chunk0 = wid * n_chunks
        row0 = chunk0 * CHUNK
        pltpu.sync_copy(xf_hbm.at[pl.ds(chunk0, n_chunks)], idx_v)

        LAG = 2
        for b in range(LAG):
            pltpu.async_copy(tp_hbm.at[idx_v.at[b]], gbuf.at[b], gsems[b])

        def outer(r, carry):
            g0 = r * NBUF
            for b in range(NBUF):
                j = g0 + b
                pltpu.make_async_copy(
                    tp_hbm.at[idx_v.at[j]], gbuf.at[b], gsems[b]
                ).wait()

                def scale_row(r2, c2):
                    for g in range(D // LN):
                        sl = pl.ds(g * LN, LN)
                        gbuf[b, r2, sl] = gbuf[b, r2, sl] * SCALE
                    return c2

                lax.fori_loop(0, CHUNK, scale_row, 0, unroll=4)

                pltpu.async_copy(
                    gbuf.at[b],
                    out_hbm.at[pl.ds(row0 + j * CHUNK, CHUNK), pl.ds(0, D)],
                    ssems[b],
                )

                bf = (b + LAG) % NBUF

                def _wait_store():
                    pltpu.make_async_copy(
                        gbuf.at[bf],
                        out_hbm.at[pl.ds(row0, CHUNK), pl.ds(0, D)],
                        ssems[bf],
                    ).wait()

                if b >= NBUF - LAG:
                    _wait_store()
                else:
                    pl.when(r > 0)(_wait_store)

                @pl.when(j + LAG < n_chunks)
                def _refill():
                    pltpu.async_copy(
                        tp_hbm.at[idx_v.at[j + LAG]], gbuf.at[bf], gsems[bf]
                    )
            return carry

        lax.fori_loop(0, n_chunks // NBUF, outer, 0)

        for j in range(n_chunks - LAG, n_chunks):
            b = j % NBUF
            pltpu.make_async_copy(
                gbuf.at[b],
                out_hbm.at[pl.ds(row0, CHUNK), pl.ds(0, D)],
                ssems[b],
            ).wait()

    return lookup


def kernel(x, table):
    B, L = x.shape
    V, D = table.shape
    N = B * L
    xf = x.reshape(N // CHUNK, CHUNK).astype(jnp.int32)
    o = _make_sc_lookup(V, N, D)(xf, table)
    return o[:, :D].reshape(B, L, D)
